# baseline (speedup 1.0000x reference)
import functools

import jax
import jax.numpy as jnp
from jax import lax
from jax.experimental import pallas as pl
from jax.experimental.pallas import tpu as pltpu
from jax.experimental.pallas import tpu_sc as plsc

VOCAB = 1000000
EMBED_DIM = 16
BATCH = 16384
HIST_LEN = 50
HPAD = 56
B = BATCH * HIST_LEN

NC = 2
NS = 16
NW = NC * NS
BATCH_PER_W = BATCH // NW
B_PER_W = B // NW
CHUNK_BATCH = 32
CHUNK = CHUNK_BATCH * HIST_LEN
NCHUNK = BATCH_PER_W // CHUNK_BATCH
NB = 2
RR = CHUNK * EMBED_DIM // 128


@functools.partial(
    pl.kernel,
    out_type=jax.ShapeDtypeStruct((B * EMBED_DIM // 128, 128), jnp.float32),
    mesh=plsc.VectorSubcoreMesh(core_axis_name="c", subcore_axis_name="s"),
    scratch_types=(
        [pltpu.VMEM((CHUNK,), jnp.int32) for _ in range(NB)]
        + [pltpu.VMEM((CHUNK, EMBED_DIM), jnp.float32) for _ in range(NB)]
        + [pltpu.VMEM((RR, 128), jnp.float32) for _ in range(NB)]
        + [pltpu.SemaphoreType.DMA for _ in range(2 * NB)]
    ),
    compiler_params=pltpu.CompilerParams(use_tc_tiling_on_sc=False),
)
def _gather_kernel(idx_hbm, table_hbm, out_hbm, *scratch):
    idx_v = scratch[0:NB]
    rows_v = scratch[NB:2 * NB]
    rows128 = scratch[2 * NB:3 * NB]
    gsem = scratch[3 * NB:4 * NB]
    osem = scratch[4 * NB:5 * NB]

    wid = lax.axis_index("s") * NC + lax.axis_index("c")
    base = wid * B_PER_W
    bbase = wid * BATCH_PER_W

    def load_idx(i):
        off = base + i * CHUNK
        pltpu.sync_copy(idx_hbm.at[pl.ds(off, CHUNK)], idx_v[i % NB])

    def start_gather(i):
        b = i % NB
        pltpu.async_copy(table_hbm.at[idx_v[b]], rows_v[b], gsem[b])

    def wait_gather(i):
        b = i % NB
        pltpu.make_async_copy(table_hbm.at[idx_v[b]], rows_v[b],
                              gsem[b]).wait()

    def repack(i):
        b = i % NB

        def row(r, carry):
            for j in range(8):
                vals = rows_v[b][r * 8 + j, :]
                rows128[b][r, pl.ds(j * EMBED_DIM, EMBED_DIM)] = vals
            return carry

        lax.fori_loop(0, RR, row, 0)

    def out_descr(i):
        b = i % NB
        r0 = wid * (B_PER_W * EMBED_DIM // 128) + i * RR
        return pltpu.make_async_copy(
            rows128[b], out_hbm.at[pl.ds(r0, RR), :], osem[b])

    def start_out(i):
        repack(i)
        out_descr(i).start()

    def wait_out(i):
        out_descr(i).wait()

    load_idx(0)
    load_idx(1)
    start_gather(0)
    start_gather(1)
    for i in range(NCHUNK):
        wait_gather(i)
        start_out(i)
        if i + 2 < NCHUNK:
            load_idx(i + 2)
            if i >= 1:
                wait_out(i - 1)
            start_gather(i + 2)
    wait_out(NCHUNK - 2)
    wait_out(NCHUNK - 1)


def kernel(inputs, embeddings):
    idx_flat = inputs.reshape(B)
    staged = _gather_kernel(idx_flat, embeddings)
    return staged.reshape(BATCH, HIST_LEN, EMBED_DIM)


# speedup vs baseline: 1.3338x; 1.3338x over previous
import functools

import jax
import jax.numpy as jnp
from jax import lax
from jax.experimental import pallas as pl
from jax.experimental.pallas import tpu as pltpu
from jax.experimental.pallas import tpu_sc as plsc

VOCAB = 1000000
EMBED_DIM = 16
BATCH = 16384
HIST_LEN = 50
HPAD = 56
B = BATCH * HIST_LEN

NC = 2
NS = 16
NW = NC * NS
BATCH_PER_W = BATCH // NW
B_PER_W = B // NW
CHUNK_BATCH = 32
CHUNK = CHUNK_BATCH * HIST_LEN
NCHUNK = BATCH_PER_W // CHUNK_BATCH
NB = 3


@functools.partial(
    pl.kernel,
    out_type=jax.ShapeDtypeStruct((BATCH * HPAD, 128), jnp.float32),
    mesh=plsc.VectorSubcoreMesh(core_axis_name="c", subcore_axis_name="s"),
    scratch_types=(
        [pltpu.VMEM((CHUNK,), jnp.int32) for _ in range(NB)]
        + [pltpu.VMEM((CHUNK, EMBED_DIM), jnp.float32) for _ in range(NB)]
        + [pltpu.SemaphoreType.DMA for _ in range(2 * NB)]
    ),
    compiler_params=pltpu.CompilerParams(use_tc_tiling_on_sc=False),
)
def _gather_kernel(idx_hbm, table_hbm, out_hbm, *scratch):
    idx_v = scratch[0:NB]
    rows_v = scratch[NB:2 * NB]
    gsem = scratch[2 * NB:3 * NB]
    osem = scratch[3 * NB:4 * NB]

    wid = lax.axis_index("s") * NC + lax.axis_index("c")
    base = wid * B_PER_W
    bbase = wid * BATCH_PER_W

    def load_idx(i):
        off = base + i * CHUNK
        pltpu.sync_copy(idx_hbm.at[pl.ds(off, CHUNK)], idx_v[i % NB])

    def start_gather(i):
        b = i % NB
        pltpu.async_copy(table_hbm.at[idx_v[b]], rows_v[b], gsem[b])

    def wait_gather(i):
        b = i % NB
        pltpu.make_async_copy(table_hbm.at[idx_v[b]], rows_v[b],
                              gsem[b]).wait()

    def out_descrs(i):
        b = i % NB
        for j in range(CHUNK_BATCH):
            bg = bbase + i * CHUNK_BATCH + j
            yield pltpu.make_async_copy(
                rows_v[b].at[pl.ds(j * HIST_LEN, HIST_LEN), :],
                out_hbm.at[pl.ds(bg * HPAD, HIST_LEN), pl.ds(0, EMBED_DIM)],
                osem[b])

    def start_out(i):
        for d in out_descrs(i):
            d.start()

    def wait_out(i):
        for d in out_descrs(i):
            d.wait()

    load_idx(0)
    load_idx(1)
    start_gather(0)
    start_gather(1)
    for i in range(NCHUNK):
        wait_gather(i)
        start_out(i)
        if i + 2 < NCHUNK:
            load_idx(i + 2)
            if i >= 1:
                wait_out(i - 1)
            start_gather(i + 2)
    wait_out(NCHUNK - 2)
    wait_out(NCHUNK - 1)


def kernel(inputs, embeddings):
    idx_flat = inputs.reshape(B)
    staged = _gather_kernel(idx_flat, embeddings)
    return staged.reshape(BATCH, HPAD, 128)[:, :HIST_LEN, :EMBED_DIM]
